# baseline (speedup 1.0000x reference)
import functools

import jax
import jax.numpy as jnp
from jax import lax
from jax.experimental import pallas as pl
from jax.experimental.pallas import tpu as pltpu
from jax.experimental.pallas import tpu_sc as plsc

B = 1024
N_CHANS = 64
WINDOW = 1000
N_OUT = 4
E = 16
FEAT = 512

WB = 40
NSTEP = WINDOW // WB
BB2 = 256


def _tc1_body(xT_ref, m_ref, acc_ref):
    i = pl.program_id(0)

    @pl.when(i == 0)
    def _():
        acc_ref[...] = jnp.zeros_like(acc_ref)

    acc_ref[...] += jnp.sum(xT_ref[...], axis=1)

    @pl.when(i == NSTEP - 1)
    def _():
        m_ref[...] = acc_ref[...] * (1.0 / WINDOW)


def _sc_gather_body(table_hbm, sid_hbm, wt_hbm, idx_v, rows_v, sem, nc):
    bpw = idx_v.shape[0]
    wid = lax.axis_index("s") * nc + lax.axis_index("c")
    base = wid * bpw
    pltpu.sync_copy(sid_hbm.at[pl.ds(base, bpw)], idx_v)
    pltpu.async_copy(table_hbm.at[idx_v], rows_v, sem).wait()
    pltpu.sync_copy(rows_v, wt_hbm.at[pl.ds(base, bpw)])


def _tc2_body(m_ref, sid_ref, Wbb_ref, bbb_ref, bh_ref, wt_ref, out_ref):
    dn = (((0,), (0,)), ((), ()))
    feats = jax.lax.dot_general(m_ref[...], Wbb_ref[...], dn,
                                preferred_element_type=jnp.float32)
    feats = jnp.maximum(feats + bbb_ref[...], 0.0)
    cols = []
    for o in range(N_OUT):
        cols.append(jnp.sum(feats * wt_ref[:, o, :], axis=1, keepdims=True))
    outv = jnp.concatenate(cols, axis=1)
    sid = sid_ref[...]
    onehot = (jax.lax.broadcasted_iota(jnp.int32, (BB2, E), 1)
              == sid).astype(jnp.float32)
    bias = jnp.dot(onehot, bh_ref[...], preferred_element_type=jnp.float32)
    out_ref[...] = outv + bias


@jax.jit
def kernel(x, subject_ids, W_bb, b_bb, W_heads, b_heads):
    xT = jnp.transpose(x, (1, 2, 0))
    sid = subject_ids.astype(jnp.int32)
    table = W_heads.transpose(0, 2, 1).reshape(E, N_OUT * FEAT)
    bbb = b_bb.reshape(1, FEAT)

    info = plsc.get_sparse_core_info()
    nc, ns = info.num_cores, info.num_subcores
    mesh = plsc.VectorSubcoreMesh(core_axis_name="c", subcore_axis_name="s")
    bpw = B // (nc * ns)
    sc_gather = pl.kernel(
        functools.partial(_sc_gather_body, nc=nc),
        mesh=mesh,
        compiler_params=pltpu.CompilerParams(use_tc_tiling_on_sc=False,
                                             needs_layout_passes=False),
        cost_estimate=pl.CostEstimate(flops=0, transcendentals=0,
                                      bytes_accessed=2 * B * N_OUT * FEAT * 4),
        out_type=jax.ShapeDtypeStruct((B, N_OUT * FEAT), jnp.float32),
        scratch_types=[
            pltpu.VMEM((bpw,), jnp.int32),
            pltpu.VMEM((bpw, N_OUT * FEAT), jnp.float32),
            pltpu.SemaphoreType.DMA,
        ],
    )
    wt = sc_gather(table, sid)
    wt3 = wt.reshape(B, N_OUT, FEAT)

    m = pl.pallas_call(
        _tc1_body,
        grid=(NSTEP,),
        in_specs=[pl.BlockSpec((N_CHANS, WB, B), lambda i: (0, i, 0))],
        out_specs=pl.BlockSpec((N_CHANS, B), lambda i: (0, 0)),
        out_shape=jax.ShapeDtypeStruct((N_CHANS, B), jnp.float32),
        scratch_shapes=[pltpu.VMEM((N_CHANS, B), jnp.float32)],
    )(xT)

    out = pl.pallas_call(
        _tc2_body,
        grid=(B // BB2,),
        in_specs=[
            pl.BlockSpec((N_CHANS, BB2), lambda j: (0, j)),
            pl.BlockSpec((BB2, 1), lambda j: (j, 0)),
            pl.BlockSpec((N_CHANS, FEAT), lambda j: (0, 0)),
            pl.BlockSpec((1, FEAT), lambda j: (0, 0)),
            pl.BlockSpec((E, N_OUT), lambda j: (0, 0)),
            pl.BlockSpec((BB2, N_OUT, FEAT), lambda j: (j, 0, 0)),
        ],
        out_specs=pl.BlockSpec((BB2, N_OUT), lambda j: (j, 0)),
        out_shape=jax.ShapeDtypeStruct((B, N_OUT), jnp.float32),
    )(m, sid.reshape(B, 1), W_bb, bbb, b_heads, wt3)
    return out


# speedup vs baseline: 1.2001x; 1.2001x over previous
import functools

import jax
import jax.numpy as jnp
from jax import lax
from jax.experimental import pallas as pl
from jax.experimental.pallas import tpu as pltpu
from jax.experimental.pallas import tpu_sc as plsc

B = 1024
N_CHANS = 64
WINDOW = 1000
N_OUT = 4
E = 16
FEAT = 512

WB = 40
NSTEP = WINDOW // WB


def _tc_body(xT_ref, Wbb_ref, bbb_ref, Wall_ref, ball_ref,
             allh_ref, acc_ref):
    i = pl.program_id(0)

    @pl.when(i == 0)
    def _():
        acc_ref[...] = jnp.zeros_like(acc_ref)

    acc_ref[...] += jnp.sum(xT_ref[...], axis=1)

    @pl.when(i == NSTEP - 1)
    def _():
        m = acc_ref[...] * (1.0 / WINDOW)
        dn = (((0,), (0,)), ((), ()))
        featsT = jax.lax.dot_general(Wbb_ref[...], m, dn,
                                     preferred_element_type=jnp.float32)
        featsT = jnp.maximum(featsT + bbb_ref[...], 0.0)
        allh = jax.lax.dot_general(featsT, Wall_ref[...], dn,
                                   preferred_element_type=jnp.float32)
        allh_ref[...] = allh + ball_ref[...]


SC_STRIPE = 128


def _sc_route_body(allh_hbm, sid_hbm, out_hbm, allh_v, sid_v, out_v, nc):
    wid = lax.axis_index("s") * nc + lax.axis_index("c")
    g = wid // N_OUT
    o = wid % N_OUT
    base = g * SC_STRIPE
    pltpu.sync_copy(allh_hbm.at[pl.ds(base * (E * N_OUT),
                                      SC_STRIPE * (E * N_OUT))], allh_v)
    pltpu.sync_copy(sid_hbm.at[pl.ds(base, SC_STRIPE)], sid_v)
    lanes = jax.lax.iota(jnp.int32, 16)
    for h in range(SC_STRIPE // 16):
        sidvec = sid_v[pl.ds(h * 16, 16)]
        idx = (lanes + h * 16) * (E * N_OUT) + sidvec * N_OUT + o
        val = plsc.load_gather(allh_v, [idx])
        out_v[pl.ds(h * 16, 16)] = val
    pltpu.sync_copy(out_v, out_hbm.at[pl.ds(o * B + base, SC_STRIPE)])


@jax.jit
def kernel(x, subject_ids, W_bb, b_bb, W_heads, b_heads):
    xT = jnp.transpose(x, (1, 2, 0))
    sid = subject_ids.astype(jnp.int32)
    W_all = W_heads.transpose(1, 0, 2).reshape(FEAT, E * N_OUT)
    b_all = b_heads.reshape(1, E * N_OUT)
    bbb = b_bb.reshape(FEAT, 1)

    allh = pl.pallas_call(
        _tc_body,
        grid=(NSTEP,),
        in_specs=[
            pl.BlockSpec((N_CHANS, WB, B), lambda i: (0, i, 0)),
            pl.BlockSpec((N_CHANS, FEAT), lambda i: (0, 0)),
            pl.BlockSpec((FEAT, 1), lambda i: (0, 0)),
            pl.BlockSpec((FEAT, E * N_OUT), lambda i: (0, 0)),
            pl.BlockSpec((1, E * N_OUT), lambda i: (0, 0)),
        ],
        out_specs=pl.BlockSpec((B, E * N_OUT), lambda i: (0, 0)),
        out_shape=jax.ShapeDtypeStruct((B, E * N_OUT), jnp.float32),
        scratch_shapes=[pltpu.VMEM((N_CHANS, B), jnp.float32)],
    )(xT, W_bb, bbb, W_all, b_all)
    allh_flat = allh.reshape(B * E * N_OUT)

    info = plsc.get_sparse_core_info()
    nc = info.num_cores
    mesh = plsc.VectorSubcoreMesh(core_axis_name="c", subcore_axis_name="s")
    sc_route = pl.kernel(
        functools.partial(_sc_route_body, nc=nc),
        mesh=mesh,
        compiler_params=pltpu.CompilerParams(use_tc_tiling_on_sc=False, needs_layout_passes=False),
        out_type=jax.ShapeDtypeStruct((N_OUT * B,), jnp.float32),
        scratch_types=[
            pltpu.VMEM((SC_STRIPE * E * N_OUT,), jnp.float32),
            pltpu.VMEM((SC_STRIPE,), jnp.int32),
            pltpu.VMEM((SC_STRIPE,), jnp.float32),
        ],
    )
    out_flat = sc_route(allh_flat, sid)
    return out_flat.reshape(N_OUT, B).T


# speedup vs baseline: 1.2334x; 1.0277x over previous
import functools

import jax
import jax.numpy as jnp
from jax import lax
from jax.experimental import pallas as pl
from jax.experimental.pallas import tpu as pltpu
from jax.experimental.pallas import tpu_sc as plsc

B = 1024
N_CHANS = 64
WINDOW = 1000
N_OUT = 4
E = 16
FEAT = 512

WB = 40
NSTEP = WINDOW // WB


def _tc_body(xT_ref, Wbb_ref, bbb_ref, Wall_ref, ball_ref,
             allh_ref, acc_ref):
    i = pl.program_id(0)

    @pl.when(i == 0)
    def _():
        acc_ref[...] = jnp.zeros_like(acc_ref)

    acc_ref[...] += jnp.sum(xT_ref[...], axis=1)

    @pl.when(i == NSTEP - 1)
    def _():
        m = acc_ref[...] * (1.0 / WINDOW)
        dn = (((0,), (0,)), ((), ()))
        featsT = jax.lax.dot_general(Wbb_ref[...], m, dn,
                                     preferred_element_type=jnp.float32)
        featsT = jnp.maximum(featsT + bbb_ref[...], 0.0)
        allh = jax.lax.dot_general(featsT, Wall_ref[...], dn,
                                   preferred_element_type=jnp.float32)
        allh = allh + ball_ref[...]
        allh_ref[...] = jnp.concatenate(
            [allh, jnp.zeros_like(allh)], axis=1)


SC_STRIPE = 128


ROW = 2 * E * N_OUT


def _sc_route_body(allh_hbm, sid_hbm, out_hbm, allh_v, sid_v, out_v, nc):
    wid = lax.axis_index("s") * nc + lax.axis_index("c")
    g = wid // N_OUT
    o = wid % N_OUT
    base = g * SC_STRIPE
    pltpu.sync_copy(allh_hbm.at[pl.ds(base * ROW, SC_STRIPE * ROW)], allh_v)
    pltpu.sync_copy(sid_hbm.at[pl.ds(base, SC_STRIPE)], sid_v)
    lanes = jax.lax.iota(jnp.int32, 16)
    for h in range(SC_STRIPE // 16):
        sidvec = sid_v[pl.ds(h * 16, 16)]
        idx = (lanes + h * 16) * ROW + sidvec * N_OUT + o
        val = plsc.load_gather(allh_v, [idx])
        out_v[pl.ds(h * 16, 16)] = val
    pltpu.sync_copy(out_v,
                    out_hbm.at[pl.ds(g * (N_OUT * SC_STRIPE) + o * SC_STRIPE,
                                     SC_STRIPE)])


@jax.jit
def kernel(x, subject_ids, W_bb, b_bb, W_heads, b_heads):
    xT = jnp.transpose(x, (1, 2, 0))
    sid = subject_ids.astype(jnp.int32)
    W_all = W_heads.transpose(1, 0, 2).reshape(FEAT, E * N_OUT)
    b_all = b_heads.reshape(1, E * N_OUT)
    bbb = b_bb.reshape(FEAT, 1)

    allh = pl.pallas_call(
        _tc_body,
        grid=(NSTEP,),
        in_specs=[
            pl.BlockSpec((N_CHANS, WB, B), lambda i: (0, i, 0)),
            pl.BlockSpec((N_CHANS, FEAT), lambda i: (0, 0)),
            pl.BlockSpec((FEAT, 1), lambda i: (0, 0)),
            pl.BlockSpec((FEAT, E * N_OUT), lambda i: (0, 0)),
            pl.BlockSpec((1, E * N_OUT), lambda i: (0, 0)),
        ],
        out_specs=pl.BlockSpec((B, ROW), lambda i: (0, 0)),
        out_shape=jax.ShapeDtypeStruct((B, ROW), jnp.float32),
        scratch_shapes=[pltpu.VMEM((N_CHANS, B), jnp.float32)],
    )(xT, W_bb, bbb, W_all, b_all)
    allh_flat = allh.reshape(B * ROW)

    info = plsc.get_sparse_core_info()
    nc = info.num_cores
    mesh = plsc.VectorSubcoreMesh(core_axis_name="c", subcore_axis_name="s")
    sc_route = pl.kernel(
        functools.partial(_sc_route_body, nc=nc),
        mesh=mesh,
        compiler_params=pltpu.CompilerParams(use_tc_tiling_on_sc=False, needs_layout_passes=False),
        out_type=jax.ShapeDtypeStruct((N_OUT * B,), jnp.float32),
        scratch_types=[
            pltpu.VMEM((SC_STRIPE * ROW,), jnp.float32),
            pltpu.VMEM((SC_STRIPE,), jnp.int32),
            pltpu.VMEM((SC_STRIPE,), jnp.float32),
        ],
    )
    out_flat = sc_route(allh_flat, sid)
    return (out_flat.reshape(B // SC_STRIPE, N_OUT, SC_STRIPE)
            .transpose(1, 0, 2).reshape(N_OUT, B).T)


# speedup vs baseline: 1.2486x; 1.0123x over previous
import functools

import jax
import jax.numpy as jnp
from jax import lax
from jax.experimental import pallas as pl
from jax.experimental.pallas import tpu as pltpu
from jax.experimental.pallas import tpu_sc as plsc

B = 1024
N_CHANS = 64
WINDOW = 1000
N_OUT = 4
E = 16
FEAT = 512

WB = 40
NSTEP = WINDOW // WB


def _tc_body(xT_ref, Wbb_ref, bbb_ref, Wall_ref, allh_ref, acc_ref):
    i = pl.program_id(0)

    @pl.when(i == 0)
    def _():
        acc_ref[...] = jnp.zeros_like(acc_ref)

    acc_ref[...] += jnp.sum(xT_ref[...], axis=1)

    @pl.when(i == NSTEP - 1)
    def _():
        m = acc_ref[...] * (1.0 / WINDOW)
        dn = (((0,), (0,)), ((), ()))
        feats = jax.lax.dot_general(m, Wbb_ref[...], dn,
                                    preferred_element_type=jnp.float32)
        feats = jnp.maximum(feats + bbb_ref[...], 0.0)
        allh = jnp.dot(feats, Wall_ref[...],
                       preferred_element_type=jnp.float32)
        allh_ref[...] = jnp.concatenate(
            [allh, jnp.zeros_like(allh)], axis=1)


SC_STRIPE = 128


ROW = 2 * E * N_OUT


def _sc_route_body(allh_hbm, sid_hbm, bh_hbm, out_hbm, allh_v, sid_v, bh_v,
                   out_v, nc):
    wid = lax.axis_index("s") * nc + lax.axis_index("c")
    g = wid // N_OUT
    o = wid % N_OUT
    base = g * SC_STRIPE
    pltpu.sync_copy(allh_hbm.at[pl.ds(base * ROW, SC_STRIPE * ROW)], allh_v)
    pltpu.sync_copy(sid_hbm.at[pl.ds(base, SC_STRIPE)], sid_v)
    pltpu.sync_copy(bh_hbm, bh_v)
    lanes = jax.lax.iota(jnp.int32, 16)
    for h in range(SC_STRIPE // 16):
        sidvec = sid_v[pl.ds(h * 16, 16)]
        idx = (lanes + h * 16) * ROW + sidvec * N_OUT + o
        val = plsc.load_gather(allh_v, [idx])
        bias = plsc.load_gather(bh_v, [sidvec * N_OUT + o])
        out_v[pl.ds(h * 16, 16)] = val + bias
    pltpu.sync_copy(out_v,
                    out_hbm.at[pl.ds(g * (N_OUT * SC_STRIPE) + o * SC_STRIPE,
                                     SC_STRIPE)])


@jax.jit
def kernel(x, subject_ids, W_bb, b_bb, W_heads, b_heads):
    xT = jnp.transpose(x, (1, 2, 0))
    sid = subject_ids.astype(jnp.int32)
    W_all = W_heads.transpose(1, 0, 2).reshape(FEAT, E * N_OUT)
    bh_flat = b_heads.reshape(E * N_OUT)
    bbb = b_bb.reshape(1, FEAT)

    allh = pl.pallas_call(
        _tc_body,
        grid=(NSTEP,),
        in_specs=[
            pl.BlockSpec((N_CHANS, WB, B), lambda i: (0, i, 0)),
            pl.BlockSpec((N_CHANS, FEAT), lambda i: (0, 0)),
            pl.BlockSpec((1, FEAT), lambda i: (0, 0)),
            pl.BlockSpec((FEAT, E * N_OUT), lambda i: (0, 0)),
        ],
        out_specs=pl.BlockSpec((B, ROW), lambda i: (0, 0)),
        out_shape=jax.ShapeDtypeStruct((B, ROW), jnp.float32),
        scratch_shapes=[pltpu.VMEM((N_CHANS, B), jnp.float32)],
    )(xT, W_bb, bbb, W_all)
    allh_flat = allh.reshape(B * ROW)

    info = plsc.get_sparse_core_info()
    nc = info.num_cores
    mesh = plsc.VectorSubcoreMesh(core_axis_name="c", subcore_axis_name="s")
    sc_route = pl.kernel(
        functools.partial(_sc_route_body, nc=nc),
        mesh=mesh,
        compiler_params=pltpu.CompilerParams(use_tc_tiling_on_sc=False, needs_layout_passes=False),
        out_type=jax.ShapeDtypeStruct((N_OUT * B,), jnp.float32),
        scratch_types=[
            pltpu.VMEM((SC_STRIPE * ROW,), jnp.float32),
            pltpu.VMEM((SC_STRIPE,), jnp.int32),
            pltpu.VMEM((E * N_OUT,), jnp.float32),
            pltpu.VMEM((SC_STRIPE,), jnp.float32),
        ],
    )
    out_flat = sc_route(allh_flat, sid, bh_flat)
    return (out_flat.reshape(B // SC_STRIPE, N_OUT, SC_STRIPE)
            .transpose(1, 0, 2).reshape(N_OUT, B).T)


# speedup vs baseline: 1.5157x; 1.2139x over previous
import jax
import jax.numpy as jnp
from jax.experimental import pallas as pl
from jax.experimental.pallas import tpu as pltpu

B = 1024
N_CHANS = 64
WINDOW = 1000
N_OUT = 4
E = 16
FEAT = 512

WB = 40
NSTEP = WINDOW // WB


def _tc_body(xT_ref, sid_ref, Wbb_ref, bbb_ref, Wall_ref, ball_ref,
             outT_ref, acc_ref):
    i = pl.program_id(0)

    @pl.when(i == 0)
    def _():
        acc_ref[...] = jnp.zeros_like(acc_ref)

    acc_ref[...] += jnp.sum(xT_ref[...], axis=1)

    @pl.when(i == NSTEP - 1)
    def _():
        m = acc_ref[...] * (1.0 / WINDOW)
        dn = (((0,), (0,)), ((), ()))
        featsT = jax.lax.dot_general(Wbb_ref[...], m, dn,
                                     preferred_element_type=jnp.float32)
        featsT = jnp.maximum(featsT + bbb_ref[...], 0.0)
        allhT = jax.lax.dot_general(Wall_ref[...], featsT, dn,
                                    preferred_element_type=jnp.float32)
        allhT = allhT + ball_ref[...]
        sid = sid_ref[...]
        row = jax.lax.broadcasted_iota(jnp.int32, (E * N_OUT, B), 0)
        mask = (row // N_OUT == sid).astype(jnp.float32)
        jo = jax.lax.broadcasted_iota(jnp.int32, (E * N_OUT, N_OUT), 0)
        oo = jax.lax.broadcasted_iota(jnp.int32, (E * N_OUT, N_OUT), 1)
        sel = (jo % N_OUT == oo).astype(jnp.float32)
        outT_ref[...] = jax.lax.dot_general(sel, allhT * mask, dn,
                                            preferred_element_type=jnp.float32)


@jax.jit
def kernel(x, subject_ids, W_bb, b_bb, W_heads, b_heads):
    xT = jnp.transpose(x, (1, 2, 0))
    sid = subject_ids.astype(jnp.int32).reshape(1, B)
    W_all = W_heads.transpose(1, 0, 2).reshape(FEAT, E * N_OUT)
    b_all = b_heads.reshape(E * N_OUT, 1)
    bbb = b_bb.reshape(FEAT, 1)
    outT = pl.pallas_call(
        _tc_body,
        grid=(NSTEP,),
        in_specs=[
            pl.BlockSpec((N_CHANS, WB, B), lambda i: (0, i, 0)),
            pl.BlockSpec((1, B), lambda i: (0, 0)),
            pl.BlockSpec((N_CHANS, FEAT), lambda i: (0, 0)),
            pl.BlockSpec((FEAT, 1), lambda i: (0, 0)),
            pl.BlockSpec((FEAT, E * N_OUT), lambda i: (0, 0)),
            pl.BlockSpec((E * N_OUT, 1), lambda i: (0, 0)),
        ],
        out_specs=pl.BlockSpec((N_OUT, B), lambda i: (0, 0)),
        out_shape=jax.ShapeDtypeStruct((N_OUT, B), jnp.float32),
        scratch_shapes=[pltpu.VMEM((N_CHANS, B), jnp.float32)],
    )(xT, sid, W_bb, bbb, W_all, b_all)
    return outT.T
